# baseline (speedup 1.0000x reference)
import functools

import jax
import jax.numpy as jnp
from jax.experimental import pallas as pl
from jax.experimental.pallas import tpu as pltpu

_ROWS = 128


def _body(mask_ref, ps_ref, pt_ref, out_ref, acc_ref, *, nsteps):
    i = pl.program_id(0)

    @pl.when(i == 0)
    def _init():
        acc_ref[0] = 0.0
        acc_ref[1] = 0.0

    m = mask_ref[...]
    ps = ps_ref[...]
    pt = pt_ref[...]
    safe = jnp.where(m > 0.5, ps, 1.0)
    acc_ref[0] += jnp.sum(pt * jnp.log(safe))
    acc_ref[1] += jnp.sum(m)

    @pl.when(i == nsteps - 1)
    def _fin():
        out_ref[...] = jnp.broadcast_to(-acc_ref[0] / acc_ref[1], (1, 1))


def kernel(ps, pt, bool_masked_pos):
    B, N, K = ps.shape
    R = B * N
    ps2 = ps.reshape(R, K)
    pt2 = pt.reshape(R, K)
    maskf = bool_masked_pos.reshape(R, 1).astype(jnp.float32)
    nsteps = pl.cdiv(R, _ROWS)
    out = pl.pallas_call(
        functools.partial(_body, nsteps=nsteps),
        grid=(nsteps,),
        in_specs=[
            pl.BlockSpec((_ROWS, 1), lambda i: (i, 0)),
            pl.BlockSpec((_ROWS, K), lambda i: (i, 0)),
            pl.BlockSpec((_ROWS, K), lambda i: (i, 0)),
        ],
        out_specs=pl.BlockSpec((1, 1), lambda i: (0, 0)),
        out_shape=jax.ShapeDtypeStruct((1, 1), jnp.float32),
        scratch_shapes=[pltpu.SMEM((2,), jnp.float32)],
    )(maskf, ps2, pt2)
    return out[0, 0]


# speedup vs baseline: 1.7504x; 1.7504x over previous
import functools

import jax
import jax.numpy as jnp
from jax.experimental import pallas as pl
from jax.experimental.pallas import tpu as pltpu


def _body(mask_ref, ps_ref, pt_ref, out_ref, acc_ref, *, nsteps):
    i = pl.program_id(0)

    @pl.when(i == 0)
    def _init():
        acc_ref[0] = 0.0
        acc_ref[1] = 0.0

    m = mask_ref[0]
    ps = ps_ref[0]
    pt = pt_ref[0]
    safe = jnp.where(m > 0.5, ps, 1.0)
    acc_ref[0] += jnp.sum(pt * jnp.log(safe))
    acc_ref[1] += jnp.sum(m)

    @pl.when(i == nsteps - 1)
    def _fin():
        out_ref[...] = jnp.broadcast_to(-acc_ref[0] / acc_ref[1], (1, 1))


def kernel(ps, pt, bool_masked_pos):
    B, N, K = ps.shape
    maskf = bool_masked_pos.astype(jnp.float32).reshape(B, N, 1)
    out = pl.pallas_call(
        functools.partial(_body, nsteps=B),
        grid=(B,),
        in_specs=[
            pl.BlockSpec((1, N, 1), lambda i: (i, 0, 0)),
            pl.BlockSpec((1, N, K), lambda i: (i, 0, 0)),
            pl.BlockSpec((1, N, K), lambda i: (i, 0, 0)),
        ],
        out_specs=pl.BlockSpec((1, 1), lambda i: (0, 0)),
        out_shape=jax.ShapeDtypeStruct((1, 1), jnp.float32),
        scratch_shapes=[pltpu.SMEM((2,), jnp.float32)],
    )(maskf, ps, pt)
    return out[0, 0]


# speedup vs baseline: 3.7961x; 2.1687x over previous
import functools

import jax
import jax.numpy as jnp
from jax.experimental import pallas as pl
from jax.experimental.pallas import tpu as pltpu


def _body(mask_ref, ps_ref, pt_ref, out_ref, acc_ref, *, nsteps):
    i = pl.program_id(0)

    @pl.when(i == 0)
    def _init():
        acc_ref[0] = 0.0
        acc_ref[1] = 0.0

    m = mask_ref[0]
    ps = ps_ref[0]
    pt = pt_ref[0]
    safe = jnp.where(m > 0.5, ps, 1.0)
    acc_ref[0] += jnp.sum(pt * jnp.log(safe))
    acc_ref[1] += jnp.sum(m)

    @pl.when(i == nsteps - 1)
    def _fin():
        out_ref[...] = jnp.broadcast_to(-acc_ref[0] / acc_ref[1], (1, 1))


def kernel(ps, pt, bool_masked_pos):
    B, N, K = ps.shape
    pst = jnp.transpose(ps, (1, 0, 2))
    ptt = jnp.transpose(pt, (1, 0, 2))
    maskf = bool_masked_pos.T.astype(jnp.float32).reshape(N, B, 1)
    out = pl.pallas_call(
        functools.partial(_body, nsteps=N),
        grid=(N,),
        in_specs=[
            pl.BlockSpec((1, B, 1), lambda i: (i, 0, 0)),
            pl.BlockSpec((1, B, K), lambda i: (i, 0, 0)),
            pl.BlockSpec((1, B, K), lambda i: (i, 0, 0)),
        ],
        out_specs=pl.BlockSpec((1, 1), lambda i: (0, 0)),
        out_shape=jax.ShapeDtypeStruct((1, 1), jnp.float32),
        scratch_shapes=[pltpu.SMEM((2,), jnp.float32)],
    )(maskf, pst, ptt)
    return out[0, 0]


# speedup vs baseline: 6.3921x; 1.6838x over previous
import functools

import jax
import jax.numpy as jnp
from jax.experimental import pallas as pl
from jax.experimental.pallas import tpu as pltpu


def _body(mask_ref, ps_ref, pt_ref, out_ref, acc_ref, *, nsteps):
    i = pl.program_id(0)

    @pl.when(i == 0)
    def _init():
        acc_ref[0] = 0.0
        acc_ref[1] = 0.0

    m = mask_ref[...]
    ps = ps_ref[...]
    pt = pt_ref[...]
    safe = jnp.where(m > 0.5, ps, 1.0)
    acc_ref[0] += jnp.sum(pt * jnp.log(safe))
    acc_ref[1] += jnp.sum(m)

    @pl.when(i == nsteps - 1)
    def _fin():
        out_ref[...] = jnp.broadcast_to(-acc_ref[0] / acc_ref[1], (1, 1))


def kernel(ps, pt, bool_masked_pos):
    B, N, K = ps.shape
    pst = jnp.transpose(ps, (1, 0, 2))
    ptt = jnp.transpose(pt, (1, 0, 2))
    maskf = bool_masked_pos.T.astype(jnp.float32).reshape(N, B, 1)
    nb = 4
    nsteps = N // nb
    out = pl.pallas_call(
        functools.partial(_body, nsteps=nsteps),
        grid=(nsteps,),
        in_specs=[
            pl.BlockSpec((nb, B, 1), lambda i: (i, 0, 0)),
            pl.BlockSpec((nb, B, K), lambda i: (i, 0, 0)),
            pl.BlockSpec((nb, B, K), lambda i: (i, 0, 0)),
        ],
        out_specs=pl.BlockSpec((1, 1), lambda i: (0, 0)),
        out_shape=jax.ShapeDtypeStruct((1, 1), jnp.float32),
        scratch_shapes=[pltpu.SMEM((2,), jnp.float32)],
    )(maskf, pst, ptt)
    return out[0, 0]


# speedup vs baseline: 6.4810x; 1.0139x over previous
import functools

import jax
import jax.numpy as jnp
from jax.experimental import pallas as pl
from jax.experimental.pallas import tpu as pltpu


def _body(mask_ref, ps_ref, pt_ref, out_ref, acc_ref, *, nsteps):
    i = pl.program_id(0)

    @pl.when(i == 0)
    def _init():
        acc_ref[0] = 0.0
        acc_ref[1] = 0.0

    m = mask_ref[...]
    ps = ps_ref[...]
    pt = pt_ref[...]
    safe = jnp.where(m > 0.5, ps, 1.0)
    acc_ref[0] += jnp.sum(pt * jnp.log(safe))
    acc_ref[1] += jnp.sum(m)

    @pl.when(i == nsteps - 1)
    def _fin():
        out_ref[...] = jnp.broadcast_to(-acc_ref[0] / acc_ref[1], (1, 1))


def kernel(ps, pt, bool_masked_pos):
    B, N, K = ps.shape
    pst = jnp.transpose(ps, (1, 0, 2))
    ptt = jnp.transpose(pt, (1, 0, 2))
    maskf = bool_masked_pos.T.astype(jnp.float32).reshape(N, B, 1)
    nb = 7
    nsteps = N // nb
    out = pl.pallas_call(
        functools.partial(_body, nsteps=nsteps),
        grid=(nsteps,),
        in_specs=[
            pl.BlockSpec((nb, B, 1), lambda i: (i, 0, 0)),
            pl.BlockSpec((nb, B, K), lambda i: (i, 0, 0)),
            pl.BlockSpec((nb, B, K), lambda i: (i, 0, 0)),
        ],
        out_specs=pl.BlockSpec((1, 1), lambda i: (0, 0)),
        out_shape=jax.ShapeDtypeStruct((1, 1), jnp.float32),
        scratch_shapes=[pltpu.SMEM((2,), jnp.float32)],
    )(maskf, pst, ptt)
    return out[0, 0]
